# Initial kernel scaffold; baseline (speedup 1.0000x reference)
import jax
import jax.numpy as jnp
from jax.experimental import pallas as pl


def kernel(p0, nonzero_idxs, t_idx, fitness):
    raise NotImplementedError("write your pallas kernel here")


# speedup vs baseline: 1.2270x; 1.2270x over previous
import functools

import jax
import jax.numpy as jnp
from jax import lax
from jax.experimental import pallas as pl
from jax.experimental.pallas import tpu as pltpu
from jax.experimental.pallas import tpu_sc as plsc

N = 16384
ROWS = 128
LANE = 128
_info = plsc.get_sparse_core_info()
_NC = _info.num_cores
_NS = _info.num_subcores
_NW = _NC * _NS
_ROWS_PER_W = ROWS // _NW


def _sc_gather_body(table_hbm, idx_hbm, out_hbm, idx_v, vals_v, sem):
    wid = lax.axis_index("s") * _NC + lax.axis_index("c")
    base = wid * _ROWS_PER_W
    pltpu.sync_copy(idx_hbm.at[pl.ds(base, _ROWS_PER_W)], idx_v)
    cps = []
    for j in range(_ROWS_PER_W):
        cps.append(pltpu.async_copy(table_hbm.at[idx_v.at[j]], vals_v.at[j], sem))
    for cp in cps:
        cp.wait()
    pltpu.sync_copy(vals_v, out_hbm.at[pl.ds(base, _ROWS_PER_W)])


_sc_gather = functools.partial(
    pl.kernel,
    mesh=plsc.VectorSubcoreMesh(core_axis_name="c", subcore_axis_name="s"),
    out_type=jax.ShapeDtypeStruct((ROWS, LANE), jnp.float32),
    scratch_types=[
        pltpu.VMEM((_ROWS_PER_W, LANE), jnp.int32),
        pltpu.VMEM((_ROWS_PER_W, LANE), jnp.float32),
        pltpu.SemaphoreType.DMA,
    ],
)(_sc_gather_body)


def _tc_math_body(t_ref, p_ref, g_ref, o_ref):
    pf = jnp.exp(g_ref[...])
    p0 = p_ref[...]
    m1 = jnp.sum(p0 * pf)
    a1 = p0 * (pf / m1)
    m2 = jnp.sum(a1 * pf)
    a2 = a1 * (pf / m2)
    sel = t_ref[0] >= 22
    o_ref[...] = jnp.log(jnp.where(sel, a2, a1))


def _tc_math(t, p0, g):
    return pl.pallas_call(
        _tc_math_body,
        out_shape=jax.ShapeDtypeStruct((ROWS, LANE), jnp.float32),
        in_specs=[
            pl.BlockSpec(memory_space=pltpu.SMEM),
            pl.BlockSpec(memory_space=pltpu.VMEM),
            pl.BlockSpec(memory_space=pltpu.VMEM),
        ],
    )(t, p0, g)


def kernel(p0, nonzero_idxs, t_idx, fitness):
    idx = nonzero_idxs.astype(jnp.int32).reshape(ROWS, LANE)
    g = _sc_gather(fitness, idx)
    t = jnp.asarray(t_idx, jnp.int32).reshape(1)
    out = _tc_math(t, p0.reshape(ROWS, LANE), g)
    return out.reshape(N)
